# Initial kernel scaffold; baseline (speedup 1.0000x reference)
import jax
import jax.numpy as jnp
from jax.experimental import pallas as pl


def kernel(x, edge_index, W1, b1, W2, b2, lin_W, lin_b):
    raise NotImplementedError("write your pallas kernel here")


# speedup vs baseline: 17.0627x; 17.0627x over previous
import functools

import jax
import jax.numpy as jnp
from jax import lax
from jax.experimental import pallas as pl
from jax.experimental.pallas import tpu as pltpu
from jax.experimental.pallas import tpu_sc as plsc

N = 10000
E = 320000
C1 = 128
COUT = 40
DEGC = 128

NC = 2
NS = 16
NW = NC * NS
EW = E // NW
K = 80
CHUNKS = EW // K
RPT = 632
NP = NS * RPT
ZR = 8

R = 1000
GRID = N // R

_mesh = functools.partial(
    plsc.VectorSubcoreMesh, core_axis_name="c", subcore_axis_name="s"
)


def _fill(ref, val):
  rows, cols = ref.shape

  def row(r, _):
    def col(q, __):
      ref[r, pl.ds(q * 16, 16)] = jnp.full((16,), val, jnp.float32)
      return __
    return lax.fori_loop(0, cols // 16, col, _)

  lax.fori_loop(0, rows, row, 0)


def _make_deg_kernel():
  @functools.partial(
      pl.kernel,
      out_type=jax.ShapeDtypeStruct((NC * NP, DEGC), jnp.float32),
      mesh=_mesh(),
      scratch_types=[
          pltpu.VMEM((CHUNKS, K), jnp.int32),
          pltpu.VMEM((K, DEGC), jnp.float32),
          pltpu.VMEM((ZR, DEGC), jnp.float32),
          pltpu.VMEM_SHARED((NP, DEGC), jnp.float32),
      ],
  )
  def deg_kernel(dst_hbm, out_hbm, dstv, ones_v, zbuf, accum):
    c = lax.axis_index("c")
    s = lax.axis_index("s")
    w = c * NS + s
    _fill(zbuf, 0.0)
    _fill(ones_v, 1.0)
    for k in range(RPT // ZR):
      pltpu.sync_copy(zbuf, accum.at[pl.ds(s * RPT + k * ZR, ZR)])
    plsc.subcore_barrier()
    pltpu.sync_copy(dst_hbm.at[w], dstv)

    def chunk(j, carry):
      pltpu.sync_copy(ones_v, accum.at[dstv.at[j]], add=True)
      return carry

    lax.fori_loop(0, CHUNKS, chunk, 0)
    plsc.subcore_barrier()
    pltpu.sync_copy(
        accum.at[pl.ds(s * RPT, RPT)],
        out_hbm.at[pl.ds(c * NP + s * RPT, RPT)],
    )

  return deg_kernel


def _make_agg_kernel(C):
  @functools.partial(
      pl.kernel,
      out_type=jax.ShapeDtypeStruct((NC * NP, C), jnp.float32),
      mesh=_mesh(),
      scratch_types=[
          pltpu.VMEM((CHUNKS, K), jnp.int32),
          pltpu.VMEM((CHUNKS, K), jnp.int32),
          pltpu.VMEM((K, C), jnp.float32),
          pltpu.VMEM((ZR, C), jnp.float32),
          pltpu.VMEM_SHARED((NP, C), jnp.float32),
          pltpu.SemaphoreType.DMA,
      ],
  )
  def agg_kernel(y_hbm, src_hbm, dst_hbm, out_hbm, srcv, dstv, rows, zbuf,
                 accum, sem):
    c = lax.axis_index("c")
    s = lax.axis_index("s")
    w = c * NS + s
    _fill(zbuf, 0.0)
    for k in range(RPT // ZR):
      pltpu.sync_copy(zbuf, accum.at[pl.ds(s * RPT + k * ZR, ZR)])
    plsc.subcore_barrier()
    pltpu.sync_copy(src_hbm.at[w], srcv)
    pltpu.sync_copy(dst_hbm.at[w], dstv)

    def chunk(j, carry):
      pltpu.async_copy(y_hbm.at[srcv.at[j]], rows, sem).wait()
      pltpu.sync_copy(rows, accum.at[dstv.at[j]], add=True)
      return carry

    lax.fori_loop(0, CHUNKS, chunk, 0)
    plsc.subcore_barrier()
    pltpu.sync_copy(
        accum.at[pl.ds(s * RPT, RPT)],
        out_hbm.at[pl.ds(c * NP + s * RPT, RPT)],
    )

  return agg_kernel


_deg = _make_deg_kernel()
_agg = _make_agg_kernel(C1)


def _tc1_body(x_ref, w1_ref, d0_ref, d1_ref, y1_ref, dinv_ref):
  deg = 1.0 + d0_ref[...] + d1_ref[...]
  dinv = lax.rsqrt(deg)
  xw = jnp.dot(x_ref[...], w1_ref[...], preferred_element_type=jnp.float32)
  y1_ref[...] = xw * dinv
  dinv_ref[...] = dinv


def _tc2_body(p0_ref, p1_ref, y1_ref, dinv_ref, b1_ref, w2_ref, y2_ref):
  dinv = dinv_ref[...]
  h1 = dinv * (p0_ref[...] + p1_ref[...] + y1_ref[...]) + b1_ref[...]
  t = jnp.dot(h1, w2_ref[...], preferred_element_type=jnp.float32)
  y2_ref[...] = t * dinv


def _tc3_body(q0_ref, q1_ref, y2_ref, dinv_ref, b2_ref, linw_ref, linb_ref,
              out_ref):
  h2 = dinv_ref[...] * (q0_ref[...] + q1_ref[...] + y2_ref[...]) + b2_ref[...]
  logits = jnp.dot(h2, linw_ref[...],
                   preferred_element_type=jnp.float32) + linb_ref[...]
  m = jnp.max(logits, axis=1, keepdims=True)
  lse = jnp.log(jnp.sum(jnp.exp(logits - m), axis=1, keepdims=True)) + m
  out_ref[...] = logits - lse


def _row_spec(cols):
  return pl.BlockSpec((R, cols), lambda i: (i, 0))


def _full_spec(rows, cols):
  return pl.BlockSpec((rows, cols), lambda i: (0, 0))


@jax.jit
def kernel(x, edge_index, W1, b1, W2, b2, lin_W, lin_b):
  ei = edge_index.astype(jnp.int32)
  src3 = ei[0].reshape(NW, CHUNKS, K)
  dst3 = ei[1].reshape(NW, CHUNKS, K)

  degp = _deg(dst3)
  d0 = degp[:N, 0:1]
  d1 = degp[NP:NP + N, 0:1]

  y1, dinv = pl.pallas_call(
      _tc1_body,
      grid=(GRID,),
      in_specs=[
          _row_spec(C1),
          _full_spec(C1, C1),
          _row_spec(1),
          _row_spec(1),
      ],
      out_specs=[_row_spec(C1), _row_spec(1)],
      out_shape=[
          jax.ShapeDtypeStruct((N, C1), jnp.float32),
          jax.ShapeDtypeStruct((N, 1), jnp.float32),
      ],
  )(x, W1, d0, d1)

  p = _agg(y1, src3, dst3)

  y2 = pl.pallas_call(
      _tc2_body,
      grid=(GRID,),
      in_specs=[
          _row_spec(C1),
          _row_spec(C1),
          _row_spec(C1),
          _row_spec(1),
          _full_spec(1, C1),
          _full_spec(C1, C1),
      ],
      out_specs=[_row_spec(C1)],
      out_shape=[jax.ShapeDtypeStruct((N, C1), jnp.float32)],
  )(p[:N], p[NP:NP + N], y1, dinv, b1.reshape(1, C1), W2)[0]

  q = _agg(y2, src3, dst3)

  out = pl.pallas_call(
      _tc3_body,
      grid=(GRID,),
      in_specs=[
          _row_spec(C1),
          _row_spec(C1),
          _row_spec(C1),
          _row_spec(1),
          _full_spec(1, C1),
          _full_spec(C1, COUT),
          _full_spec(1, COUT),
      ],
      out_specs=[_row_spec(COUT)],
      out_shape=[jax.ShapeDtypeStruct((N, COUT), jnp.float32)],
  )(q[:N], q[NP:NP + N], y2, dinv, b2.reshape(1, C1), lin_W,
    lin_b.reshape(1, COUT))[0]

  return out
